# baseline (speedup 1.0000x reference)
import functools

import jax
import jax.numpy as jnp
from jax import lax
from jax.experimental import pallas as pl
from jax.experimental.pallas import tpu as pltpu
from jax.experimental.pallas import tpu_sc as plsc

B = 16384
D = 64
NC = 2
NS = 16
NW = NC * NS
PER_W = B // NW
C = 64
NCHUNK = PER_W // C
GROUPS = C // 16
NROWS = B // C
ROW_UNROLL = 2


def _neg_sqrt(x):
    xi = lax.bitcast_convert_type(x, jnp.int32)
    y = lax.bitcast_convert_type(jnp.int32(0x5F3759DF) - (xi >> 1), jnp.float32)
    half = jnp.float32(0.5) * x
    for _ in range(3):
        y = y * (jnp.float32(1.5) - half * y * y)
    return -(x * y)


def _body(s_hbm, r_hbm, o_hbm, t_hbm, par_hbm, ent, rel, tim, out,
          sidx, ridx, oidx, tidx,
          sb0, sb1, rb0, rb1, ob0, ob1, tb0, tb1,
          psum, outv, par_vm, sem0, sem1):
    cid = lax.axis_index("c")
    sid = lax.axis_index("s")
    wid = sid * NC + cid
    row0 = wid * NCHUNK

    pltpu.sync_copy(s_hbm.at[pl.ds(row0, NCHUNK)], sidx)
    pltpu.sync_copy(r_hbm.at[pl.ds(row0, NCHUNK)], ridx)
    pltpu.sync_copy(o_hbm.at[pl.ds(row0, NCHUNK)], oidx)
    pltpu.sync_copy(t_hbm.at[pl.ds(row0, NCHUNK)], tidx)
    pltpu.sync_copy(par_hbm.at[pl.ds(row0, NCHUNK)], par_vm)

    sbufs = (sb0, sb1)
    rbufs = (rb0, rb1)
    obufs = (ob0, ob1)
    tbufs = (tb0, tb1)
    sems = (sem0, sem1)

    def fire(chunk):
        slot = chunk % 2
        sem = sems[slot]
        return (
            pltpu.async_copy(ent.at[sidx.at[chunk]], sbufs[slot], sem),
            pltpu.async_copy(rel.at[ridx.at[chunk]], rbufs[slot], sem),
            pltpu.async_copy(ent.at[oidx.at[chunk]], obufs[slot], sem),
            pltpu.async_copy(tim.at[tidx.at[chunk]], tbufs[slot], sem),
        )

    handles = {0: fire(0)}

    for chunk in range(NCHUNK):
        slot = chunk % 2
        if chunk + 1 < NCHUNK:
            handles[chunk + 1] = fire(chunk + 1)
        for h in handles.pop(chunk):
            h.wait()

        sb, rb, ob, tb = sbufs[slot], rbufs[slot], obufs[slot], tbufs[slot]

        def row_body(g, _, sb=sb, rb=rb, ob=ob, tb=tb, chunk=chunk):
            pv = par_vm[chunk, pl.ds(g * 16, 16)]
            for u in range(16):
                cc = g * 16 + u
                p = pv[u]
                offs = (p & 1) * 64
                offr = ((p >> 1) & 1) * 64
                offo = ((p >> 2) & 1) * 64
                offt = ((p >> 3) & 1) * 64
                acc = jnp.zeros((16,), jnp.float32)
                for k in range(D // 16):
                    dv = ((sb[cc, pl.ds(offs + k * 16, 16)]
                           + rb[cc, pl.ds(offr + k * 16, 16)])
                          + (tb[cc, pl.ds(offt + k * 16, 16)]
                             - ob[cc, pl.ds(offo + k * 16, 16)]))
                    acc = acc + dv * dv
                psum[pl.ds(cc * 17, 16)] = acc
            return 0

        lax.fori_loop(0, GROUPS, row_body, 0)

        def group_body(g, _, chunk=chunk):
            rows = lax.iota(jnp.int32, 16) + g * 16
            base = rows * 17
            tot = jnp.zeros((16,), jnp.float32)
            for j in range(16):
                tot = tot + plsc.load_gather(psum, [base + j])
            outv[pl.ds(chunk * C + g * 16, 16)] = _neg_sqrt(tot)
            return 0

        lax.fori_loop(0, GROUPS, group_body, 0)

    pltpu.sync_copy(outv, out.at[pl.ds(wid * PER_W, PER_W)])


_ttranse = functools.partial(
    pl.kernel,
    out_type=jax.ShapeDtypeStruct((B,), jnp.float32),
    mesh=plsc.VectorSubcoreMesh(core_axis_name="c", subcore_axis_name="s"),
    compiler_params=pltpu.CompilerParams(
        needs_layout_passes=False, use_tc_tiling_on_sc=True),
    scratch_types=[
        pltpu.VMEM((NCHUNK, C), jnp.int32),
        pltpu.VMEM((NCHUNK, C), jnp.int32),
        pltpu.VMEM((NCHUNK, C), jnp.int32),
        pltpu.VMEM((NCHUNK, C), jnp.int32),
        pltpu.VMEM((C, 2 * D), jnp.float32),
        pltpu.VMEM((C, 2 * D), jnp.float32),
        pltpu.VMEM((C, 2 * D), jnp.float32),
        pltpu.VMEM((C, 2 * D), jnp.float32),
        pltpu.VMEM((C, 2 * D), jnp.float32),
        pltpu.VMEM((C, 2 * D), jnp.float32),
        pltpu.VMEM((C, 2 * D), jnp.float32),
        pltpu.VMEM((C, 2 * D), jnp.float32),
        pltpu.VMEM((C * 17,), jnp.float32),
        pltpu.VMEM((PER_W,), jnp.float32),
        pltpu.VMEM((NCHUNK, C), jnp.int32),
        pltpu.SemaphoreType.DMA,
        pltpu.SemaphoreType.DMA,
    ],
)(_body)


TCOLS = 4096
NTBLK = (1000000 + TCOLS - 1) // TCOLS
EROWS = NTBLK * (TCOLS // 2)


def _tr_body(in_ref, out_ref):
    x = in_ref[...]
    eye = jnp.float32(
        lax.broadcasted_iota(jnp.int32, (D, D), 0)
        == lax.broadcasted_iota(jnp.int32, (D, D), 1))
    y = lax.dot_general(x, eye, (((0,), (0,)), ((), ())),
                        preferred_element_type=jnp.float32)
    h = TCOLS // 2
    out_ref[...] = jnp.concatenate([y[:h], y[h:]], axis=1)


def _transpose(ent_t):
    return pl.pallas_call(
        _tr_body,
        out_shape=jax.ShapeDtypeStruct((EROWS, 2 * D), jnp.float32),
        grid=(NTBLK,),
        in_specs=[pl.BlockSpec((D, TCOLS), lambda i: (0, i))],
        out_specs=pl.BlockSpec((TCOLS // 2, 2 * D), lambda i: (i, 0)),
    )(ent_t)


def kernel(input_0, input_1, input_2, input_3, entities, relations, times):
    s = input_0.astype(jnp.int32)
    r = input_1.astype(jnp.int32)
    o = input_2.astype(jnp.int32)
    t = input_3.astype(jnp.int32)
    par = (((s >> 11) & 1) | ((r & 1) << 1) | (((o >> 11) & 1) << 2)
           | ((t & 1) << 3))
    return _ttranse(
        ((s >> 12) * 2048 + (s & 2047)).reshape(NROWS, C),
        (r >> 1).reshape(NROWS, C),
        ((o >> 12) * 2048 + (o & 2047)).reshape(NROWS, C),
        (t >> 1).reshape(NROWS, C),
        par.reshape(NROWS, C),
        _transpose(entities.T),
        relations.reshape(500, 2 * D),
        times.reshape(500, 2 * D),
    )


# speedup vs baseline: 1.2217x; 1.2217x over previous
import functools

import jax
import jax.numpy as jnp
from jax import lax
from jax.experimental import pallas as pl
from jax.experimental.pallas import tpu as pltpu
from jax.experimental.pallas import tpu_sc as plsc

B = 16384
D = 64
NC = 2
NS = 16
NW = NC * NS
PER_W = B // NW
C = 64
NCHUNK = PER_W // C
GROUPS = C // 16
NROWS = B // C
ROW_UNROLL = 2


def _neg_sqrt(x):
    xi = lax.bitcast_convert_type(x, jnp.int32)
    y = lax.bitcast_convert_type(jnp.int32(0x5F3759DF) - (xi >> 1), jnp.float32)
    half = jnp.float32(0.5) * x
    for _ in range(3):
        y = y * (jnp.float32(1.5) - half * y * y)
    return -(x * y)


def _body(s_hbm, r_hbm, o_hbm, t_hbm, par_hbm, ent, rel, tim, out,
          sidx, ridx, oidx, tidx,
          sb0, sb1, rb0, rb1, ob0, ob1, tb0, tb1,
          psum, outv, par_vm, sem0, sem1):
    cid = lax.axis_index("c")
    sid = lax.axis_index("s")
    wid = sid * NC + cid
    row0 = wid * NCHUNK

    pltpu.sync_copy(s_hbm.at[pl.ds(row0, NCHUNK)], sidx)
    pltpu.sync_copy(r_hbm.at[pl.ds(row0, NCHUNK)], ridx)
    pltpu.sync_copy(o_hbm.at[pl.ds(row0, NCHUNK)], oidx)
    pltpu.sync_copy(t_hbm.at[pl.ds(row0, NCHUNK)], tidx)
    pltpu.sync_copy(par_hbm.at[pl.ds(row0, NCHUNK)], par_vm)

    sbufs = (sb0, sb1)
    rbufs = (rb0, rb1)
    obufs = (ob0, ob1)
    tbufs = (tb0, tb1)
    sems = (sem0, sem1)

    def fire(chunk):
        slot = chunk % 2
        sem = sems[slot]
        return (
            pltpu.async_copy(ent.at[sidx.at[chunk]], sbufs[slot], sem),
            pltpu.async_copy(rel.at[ridx.at[chunk]], rbufs[slot], sem),
            pltpu.async_copy(ent.at[oidx.at[chunk]], obufs[slot], sem),
            pltpu.async_copy(tim.at[tidx.at[chunk]], tbufs[slot], sem),
        )

    handles = {0: fire(0)}

    for chunk in range(NCHUNK):
        slot = chunk % 2
        if chunk + 1 < NCHUNK:
            handles[chunk + 1] = fire(chunk + 1)
        for h in handles.pop(chunk):
            h.wait()

        sb, rb, ob, tb = sbufs[slot], rbufs[slot], obufs[slot], tbufs[slot]

        def row_body(g, _, sb=sb, rb=rb, ob=ob, tb=tb, chunk=chunk):
            pv = par_vm[chunk, pl.ds(g * 16, 16)]
            for u in range(16):
                cc = g * 16 + u
                p = pv[u]
                offs = (p & 1) * 64
                offr = ((p >> 1) & 1) * 64
                offo = ((p >> 2) & 1) * 64
                offt = ((p >> 3) & 1) * 64
                acc = jnp.zeros((16,), jnp.float32)
                for k in range(D // 16):
                    dv = ((sb[cc, pl.ds(offs + k * 16, 16)]
                           + rb[cc, pl.ds(offr + k * 16, 16)])
                          + (tb[cc, pl.ds(offt + k * 16, 16)]
                             - ob[cc, pl.ds(offo + k * 16, 16)]))
                    acc = acc + dv * dv
                psum[pl.ds(cc * 17, 16)] = acc
            return 0

        lax.fori_loop(0, GROUPS, row_body, 0)

        def group_body(g, _, chunk=chunk):
            rows = lax.iota(jnp.int32, 16) + g * 16
            base = rows * 17
            tot = jnp.zeros((16,), jnp.float32)
            for j in range(16):
                tot = tot + plsc.load_gather(psum, [base + j])
            outv[pl.ds(chunk * C + g * 16, 16)] = _neg_sqrt(tot)
            return 0

        lax.fori_loop(0, GROUPS, group_body, 0)

    pltpu.sync_copy(outv, out.at[pl.ds(wid * PER_W, PER_W)])


_ttranse = functools.partial(
    pl.kernel,
    out_type=jax.ShapeDtypeStruct((B,), jnp.float32),
    mesh=plsc.VectorSubcoreMesh(core_axis_name="c", subcore_axis_name="s"),
    compiler_params=pltpu.CompilerParams(
        needs_layout_passes=False, use_tc_tiling_on_sc=True),
    scratch_types=[
        pltpu.VMEM((NCHUNK, C), jnp.int32),
        pltpu.VMEM((NCHUNK, C), jnp.int32),
        pltpu.VMEM((NCHUNK, C), jnp.int32),
        pltpu.VMEM((NCHUNK, C), jnp.int32),
        pltpu.VMEM((C, 2 * D), jnp.float32),
        pltpu.VMEM((C, 2 * D), jnp.float32),
        pltpu.VMEM((C, 2 * D), jnp.float32),
        pltpu.VMEM((C, 2 * D), jnp.float32),
        pltpu.VMEM((C, 2 * D), jnp.float32),
        pltpu.VMEM((C, 2 * D), jnp.float32),
        pltpu.VMEM((C, 2 * D), jnp.float32),
        pltpu.VMEM((C, 2 * D), jnp.float32),
        pltpu.VMEM((C * 17,), jnp.float32),
        pltpu.VMEM((PER_W,), jnp.float32),
        pltpu.VMEM((NCHUNK, C), jnp.int32),
        pltpu.SemaphoreType.DMA,
        pltpu.SemaphoreType.DMA,
    ],
)(_body)


TCOLS = 8192
NTBLK = (1000000 + TCOLS - 1) // TCOLS
EROWS = NTBLK * (TCOLS // 2)


def _tr_body(in_ref, out_ref):
    y = in_ref[...].T
    h = TCOLS // 2
    out_ref[...] = jnp.concatenate([y[:h], y[h:]], axis=1)


def _transpose(ent_t):
    return pl.pallas_call(
        _tr_body,
        out_shape=jax.ShapeDtypeStruct((EROWS, 2 * D), jnp.float32),
        grid=(NTBLK,),
        in_specs=[pl.BlockSpec((D, TCOLS), lambda i: (0, i))],
        out_specs=pl.BlockSpec((TCOLS // 2, 2 * D), lambda i: (i, 0)),
    )(ent_t)


def kernel(input_0, input_1, input_2, input_3, entities, relations, times):
    s = input_0.astype(jnp.int32)
    r = input_1.astype(jnp.int32)
    o = input_2.astype(jnp.int32)
    t = input_3.astype(jnp.int32)
    hb = TCOLS // 2
    par = (((s // hb) & 1) | ((r & 1) << 1) | (((o // hb) & 1) << 2)
           | ((t & 1) << 3))
    return _ttranse(
        ((s // TCOLS) * hb + (s % hb)).reshape(NROWS, C),
        (r >> 1).reshape(NROWS, C),
        ((o // TCOLS) * hb + (o % hb)).reshape(NROWS, C),
        (t >> 1).reshape(NROWS, C),
        par.reshape(NROWS, C),
        _transpose(entities.T),
        relations.reshape(500, 2 * D),
        times.reshape(500, 2 * D),
    )
